# baseline (speedup 1.0000x reference)
import jax
import jax.numpy as jnp
from jax.experimental import pallas as pl


_B0 = 4
_B1 = 4096
_COLS = 4096
_BLK = 256


def _body(in_ref, out_ref):
    out_ref[...] = in_ref[...].reshape(1, _BLK, _COLS)


def kernel(tensor):
    out = pl.pallas_call(
        _body,
        grid=(_B0, _B1 // _BLK),
        in_specs=[pl.BlockSpec((1, _BLK, 32, 128), lambda i, j: (i, j, 0, 0))],
        out_specs=pl.BlockSpec((1, _BLK, _COLS), lambda i, j: (i, j, 0)),
        out_shape=jax.ShapeDtypeStruct((_B0, _B1, _COLS), jnp.float32),
    )(tensor)
    return out


# speedup vs baseline: 1.0236x; 1.0236x over previous
import jax
import jax.numpy as jnp
from jax.experimental import pallas as pl


_B0 = 4
_B1 = 4096
_COLS = 4096
_BLK = 512


def _body(in_ref, out_ref):
    out_ref[...] = in_ref[...].reshape(1, _BLK, _COLS)


def kernel(tensor):
    out = pl.pallas_call(
        _body,
        grid=(_B0, _B1 // _BLK),
        in_specs=[pl.BlockSpec((1, _BLK, 32, 128), lambda i, j: (i, j, 0, 0))],
        out_specs=pl.BlockSpec((1, _BLK, _COLS), lambda i, j: (i, j, 0)),
        out_shape=jax.ShapeDtypeStruct((_B0, _B1, _COLS), jnp.float32),
    )(tensor)
    return out
